# baseline (speedup 1.0000x reference)
import functools

import jax
import jax.numpy as jnp
from jax import lax
from jax.experimental import pallas as pl
from jax.experimental.pallas import tpu as pltpu
from jax.experimental.pallas import tpu_sc as plsc

N = 10000
E = 320000
NC = 2
NS = 16
LANES = 16
CHUNK = 128
G = 32
NGRP = 5
NBUF = 2
NCH = G * NGRP
E_PAD = NS * NCH * CHUNK
DCH = E_PAD // (NC * NS * CHUNK)
N_ACC = 10240
ZROWS = N_ACC // NS
OCH = ZROWS // CHUNK

@functools.lru_cache(maxsize=None)
def _mesh():
    return plsc.VectorSubcoreMesh(core_axis_name="c", subcore_axis_name="s",
                                  num_cores=NC, num_subcores=NS)


def _deg_body(dst_hbm, out_hbm, idx_v, ones_v, stage_v, acc_sh):
    c = lax.axis_index("c")
    s = lax.axis_index("s")

    @pl.loop(0, CHUNK)
    def _(r):
        ones_v[pl.ds(r, 1), pl.ds(0, LANES)] = jnp.ones((1, LANES), jnp.float32)
        stage_v[pl.ds(r, 1), pl.ds(0, LANES)] = jnp.zeros((1, LANES), jnp.float32)

    @pl.loop(0, ZROWS // CHUNK)
    def _(j):
        pltpu.sync_copy(stage_v, acc_sh.at[pl.ds(s * ZROWS + j * CHUNK, CHUNK)])

    plsc.subcore_barrier()
    pltpu.sync_copy(dst_hbm.at[c, s], idx_v)

    @pl.loop(0, DCH)
    def _(j):
        pltpu.sync_copy(ones_v, acc_sh.at[idx_v.at[j]], add=True)

    plsc.subcore_barrier()

    @pl.loop(0, OCH)
    def _(j):
        base = s * ZROWS + j * CHUNK
        pltpu.sync_copy(acc_sh.at[pl.ds(base, CHUNK)], stage_v)
        pltpu.sync_copy(stage_v, out_hbm.at[c, pl.ds(base, CHUNK)])


@functools.lru_cache(maxsize=None)
def _deg_call():
  return pl.kernel(
    _deg_body,
    out_type=jax.ShapeDtypeStruct((NC, N_ACC, LANES), jnp.float32),
    mesh=_mesh(),
    compiler_params=pltpu.CompilerParams(use_tc_tiling_on_sc=False),
    scratch_types=[
        pltpu.VMEM((DCH, CHUNK), jnp.int32),
        pltpu.VMEM((CHUNK, LANES), jnp.float32),
        pltpu.VMEM((CHUNK, LANES), jnp.float32),
        pltpu.VMEM_SHARED((N_ACC, LANES), jnp.float32),
    ],
  )


@functools.lru_cache(maxsize=None)
def _prop_call(fh):
    def body(*refs):
        table_hbm, src_hbm, dst_hbm, out_hbm, src_v, dst_v = refs[:6]
        bufs = refs[6:6 + NBUF]
        acc_sh = refs[6 + NBUF]
        gsems = refs[7 + NBUF:7 + 2 * NBUF]
        ssems = refs[7 + 2 * NBUF:7 + 3 * NBUF]
        c = lax.axis_index("c")
        s = lax.axis_index("s")

        @pl.loop(0, CHUNK)
        def _(r):
            @pl.loop(0, fh // LANES)
            def _(k):
                bufs[0][pl.ds(r, 1), pl.ds(k * LANES, LANES)] = (
                    jnp.zeros((1, LANES), jnp.float32))

        @pl.loop(0, OCH)
        def _(j):
            pltpu.async_copy(
                bufs[0], acc_sh.at[pl.ds(s * ZROWS + j * CHUNK, CHUNK)], ssems[0])

        @pl.loop(0, OCH)
        def _(j):
            pltpu.make_async_copy(
                bufs[0], acc_sh.at[pl.ds(0, CHUNK)], ssems[0]).wait()

        plsc.subcore_barrier()

        @pl.loop(0, NGRP)
        def _(g):
            pltpu.sync_copy(src_hbm.at[c, s, g], src_v)
            pltpu.sync_copy(dst_hbm.at[s, g], dst_v)
            for b in range(NBUF):
                pltpu.async_copy(table_hbm.at[src_v.at[b]], bufs[b], gsems[b])

            @pl.loop(0, G // NBUF)
            def _(q):
                j = q * NBUF
                for b in range(NBUF):
                    pltpu.make_async_copy(
                        table_hbm.at[src_v.at[j + b]], bufs[b], gsems[b]).wait()
                    pltpu.async_copy(
                        bufs[b], acc_sh.at[dst_v.at[j + b]], ssems[b], add=True)
                for b in range(NBUF):
                    pltpu.make_async_copy(
                        bufs[b], acc_sh.at[dst_v.at[0]], ssems[b]).wait()

                    @pl.when(j + b + NBUF < G)
                    def _():
                        pltpu.async_copy(
                            table_hbm.at[src_v.at[j + b + NBUF]], bufs[b],
                            gsems[b])

        plsc.subcore_barrier()

        for k in range(OCH):
            b = k % 2
            if k >= 2:
                pltpu.make_async_copy(
                    bufs[b], out_hbm.at[c, pl.ds(0, CHUNK)], gsems[b]).wait()
            base = s * ZROWS + k * CHUNK
            pltpu.sync_copy(acc_sh.at[pl.ds(base, CHUNK)], bufs[b])
            pltpu.async_copy(bufs[b], out_hbm.at[c, pl.ds(base, CHUNK)], gsems[b])
        for k in range(max(0, OCH - 2), OCH):
            pltpu.make_async_copy(
                bufs[k % 2], out_hbm.at[c, pl.ds(0, CHUNK)], gsems[k % 2]).wait()

    return pl.kernel(
        body,
        out_type=jax.ShapeDtypeStruct((NC, N_ACC, fh), jnp.float32),
        mesh=_mesh(),
        compiler_params=pltpu.CompilerParams(use_tc_tiling_on_sc=False),
        scratch_types=(
            [pltpu.VMEM((G, CHUNK), jnp.int32),
             pltpu.VMEM((G, CHUNK), jnp.int32)]
            + [pltpu.VMEM((CHUNK, fh), jnp.float32)] * NBUF
            + [pltpu.VMEM_SHARED((N_ACC, fh), jnp.float32)]
            + [pltpu.SemaphoreType.DMA] * (2 * NBUF)
        ),
    )


RB = 400
GRID = N // RB


def _dinv_block(deg_ref):
    dsum = deg_ref[0, :, 0:1] + deg_ref[1, :, 0:1] + 1.0
    return lax.rsqrt(jnp.maximum(dsum, 1.0))


def _tc_first(x, w, deg):
    f_in, f_out = w.shape
    fh = f_out // 2

    def body(x_ref, w_ref, deg_ref, out_ref):
        dinv = _dinv_block(deg_ref)
        v = jnp.dot(x_ref[...], w_ref[...], preferred_element_type=jnp.float32)
        vs = v * dinv
        out_ref[0] = vs[:, :fh]
        out_ref[1] = vs[:, fh:]

    return pl.pallas_call(
        body,
        grid=(GRID,),
        in_specs=[
            pl.BlockSpec((RB, f_in), lambda i: (i, 0)),
            pl.BlockSpec((f_in, f_out), lambda i: (0, 0)),
            pl.BlockSpec((NC, RB, LANES), lambda i: (0, i, 0)),
        ],
        out_specs=pl.BlockSpec((NC, RB, fh), lambda i: (0, i, 0)),
        out_shape=jax.ShapeDtypeStruct((NC, N, fh), jnp.float32),
    )(x, w, deg)


def _tc_mid(acc, table, deg, b, w):
    f_in, f_out = w.shape
    f2 = f_in // 2
    fh = f_out // 2

    def body(acc_ref, tab_ref, deg_ref, b_ref, w_ref, out_ref):
        dinv = _dinv_block(deg_ref)
        bb = b_ref[...]
        u0 = jnp.maximum((acc_ref[0] + tab_ref[0]) * dinv + bb[:, :f2], 0.0)
        u1 = jnp.maximum((acc_ref[1] + tab_ref[1]) * dinv + bb[:, f2:], 0.0)
        v = (jnp.dot(u0, w_ref[0:f2, :], preferred_element_type=jnp.float32)
             + jnp.dot(u1, w_ref[f2:, :], preferred_element_type=jnp.float32))
        vs = v * dinv
        out_ref[0] = vs[:, :fh]
        out_ref[1] = vs[:, fh:]

    return pl.pallas_call(
        body,
        grid=(GRID,),
        in_specs=[
            pl.BlockSpec((NC, RB, f2), lambda i: (0, i, 0)),
            pl.BlockSpec((NC, RB, f2), lambda i: (0, i, 0)),
            pl.BlockSpec((NC, RB, LANES), lambda i: (0, i, 0)),
            pl.BlockSpec((1, f_in), lambda i: (0, 0)),
            pl.BlockSpec((f_in, f_out), lambda i: (0, 0)),
        ],
        out_specs=pl.BlockSpec((NC, RB, fh), lambda i: (0, i, 0)),
        out_shape=jax.ShapeDtypeStruct((NC, N, fh), jnp.float32),
    )(acc, table, deg, b, w)


def _tc_last(acc, table, deg, b, wr, br):
    f_in = wr.shape[0]
    f2 = f_in // 2

    def body(acc_ref, tab_ref, deg_ref, b_ref, w_ref, br_ref, out_ref):
        dinv = _dinv_block(deg_ref)
        bb = b_ref[...]
        u0 = jnp.maximum((acc_ref[0] + tab_ref[0]) * dinv + bb[:, :f2], 0.0)
        u1 = jnp.maximum((acc_ref[1] + tab_ref[1]) * dinv + bb[:, f2:], 0.0)
        v = (jnp.dot(u0, w_ref[0:f2, :], preferred_element_type=jnp.float32)
             + jnp.dot(u1, w_ref[f2:, :], preferred_element_type=jnp.float32))
        out_ref[...] = v + br_ref[...]

    return pl.pallas_call(
        body,
        grid=(GRID,),
        in_specs=[
            pl.BlockSpec((NC, RB, f2), lambda i: (0, i, 0)),
            pl.BlockSpec((NC, RB, f2), lambda i: (0, i, 0)),
            pl.BlockSpec((NC, RB, LANES), lambda i: (0, i, 0)),
            pl.BlockSpec((1, f_in), lambda i: (0, 0)),
            pl.BlockSpec((f_in, 1), lambda i: (0, 0)),
            pl.BlockSpec((1, 1), lambda i: (0, 0)),
        ],
        out_specs=pl.BlockSpec((RB, 1), lambda i: (i, 0)),
        out_shape=jax.ShapeDtypeStruct((N, 1), jnp.float32),
    )(acc, table, deg, b, wr, br)


def kernel(x, edge_index, W0, b0, W1, b1, W2, b2, W3, b3, W4, b4,
           W5, b5, W6, b6, W7, b7, Wr, br):
    src = edge_index[0]
    dst = edge_index[1]
    pad = E_PAD - E
    src_p = jnp.concatenate([src, jnp.zeros((pad,), src.dtype)])
    pad_dst = N + jnp.arange(pad, dtype=dst.dtype) % (N_ACC - N)
    dst_p = jnp.concatenate([dst, pad_dst])
    src2 = jnp.stack([src_p, src_p + N]).reshape(NC, NS, NGRP, G, CHUNK)
    dst_prop = dst_p.reshape(NS, NGRP, G, CHUNK)
    dst_deg = dst_p.reshape(NC, NS, DCH, CHUNK)

    deg = _deg_call()(dst_deg)

    Ws = [W1, W2, W3, W4, W5, W6, W7]
    bs = [b0, b1, b2, b3, b4, b5, b6]

    table = _tc_first(x, W0, deg)
    for l in range(7):
        fh = table.shape[2]
        acc = _prop_call(fh)(table.reshape(NC * N, fh), src2, dst_prop)
        table = _tc_mid(acc, table, deg, bs[l].reshape(1, -1), Ws[l])

    fh = table.shape[2]
    acc = _prop_call(fh)(table.reshape(NC * N, fh), src2, dst_prop)
    return _tc_last(acc, table, deg, b7.reshape(1, -1), Wr, br.reshape(1, 1))
